# baseline (speedup 1.0000x reference)
import jax
import jax.numpy as jnp
from jax import lax
from jax.experimental import pallas as pl
from jax.experimental.pallas import tpu as pltpu
from jax.experimental.pallas import tpu_sc as plsc

_B = 8
_N = 784
_L = 256
_D = 10000
_BLK = 128
_NBLK = 79
_NC = 112
_NSUB = 32


def _quantize_chunk(v):
    y = v * jnp.float32(_L - 1)
    i = y.astype(jnp.int32)
    frac = y - i.astype(jnp.float32)
    half = jnp.float32(0.5)
    odd = (i & 1) == 1
    inc = (frac > half) | ((frac == half) & odd)
    idx = i + jnp.where(inc, jnp.int32(1), jnp.int32(0))
    return jnp.minimum(jnp.maximum(idx, 0), _L - 1)


def _sc_body(x_hbm, p_hbm, lw_hbm, pt_hbm, lwt_hbm, out_hbm,
             x_v, idx_v, lw_v, p_v, acc_v):
    c = lax.axis_index("c")
    s = lax.axis_index("s")
    wid = s * 2 + c

    pltpu.sync_copy(x_hbm, x_v)

    def qbody(t, carry):
        v = x_v[pl.ds(t * 16, 16)]
        idx_v[pl.ds(t * 16, 16)] = _quantize_chunk(v)
        return carry

    lax.fori_loop(0, (_B * _N) // 16, qbody, 0)

    zero16 = jnp.zeros((16,), jnp.float32)

    def blk_body(i, carry):
        blk = wid + _NSUB * i
        is_tail = blk == (_NBLK - 1)
        c0 = pl.multiple_of(blk * _BLK, _BLK)

        @pl.when(blk < _NBLK)
        def _process():
            @pl.when(is_tail)
            def _():
                pltpu.sync_copy(lwt_hbm, lw_v)

            @pl.when(jnp.logical_not(is_tail))
            def _():
                pltpu.sync_copy(lw_hbm.at[:, pl.ds(c0, _BLK)], lw_v)

            def zbody(g, carry):
                for b in range(_B):
                    acc_v[b, pl.ds(g * 16, 16)] = zero16
                return carry

            lax.fori_loop(0, _BLK // 16, zbody, 0)

            def nc_body(nc, carry):
                n0 = pl.multiple_of(nc * _NC, _NC)

                @pl.when(is_tail)
                def _():
                    pltpu.sync_copy(pt_hbm.at[pl.ds(n0, _NC), :], p_v)

                @pl.when(jnp.logical_not(is_tail))
                def _():
                    pltpu.sync_copy(
                        p_hbm.at[pl.ds(n0, _NC), pl.ds(c0, _BLK)], p_v)

                for half in range(2):
                    h0 = half * 64
                    acc = tuple(
                        acc_v[b, pl.ds(h0 + k * 16, 16)]
                        for b in range(_B) for k in range(4))

                    def nbody(t, acc, _n0=n0, _h0=h0):
                        acc = list(acc)
                        iv = idx_v[pl.ds((_n0 + 2 * t) * _B, 16)]
                        for j in range(2):
                            nl = 2 * t + j
                            pvec = [p_v[nl, pl.ds(_h0 + k * 16, 16)]
                                    for k in range(4)]
                            for b in range(_B):
                                sidx = iv[j * _B + b]
                                for k in range(4):
                                    lw = lw_v[sidx, pl.ds(_h0 + k * 16, 16)]
                                    acc[b * 4 + k] = (
                                        acc[b * 4 + k] + pvec[k] * lw)
                        return tuple(acc)

                    acc = lax.fori_loop(0, _NC // 2, nbody, acc)

                    for b in range(_B):
                        for k in range(4):
                            acc_v[b, pl.ds(h0 + k * 16, 16)] = acc[b * 4 + k]
                return carry

            lax.fori_loop(0, _N // _NC, nc_body, 0)

            one = jnp.float32(1.0)

            def sgn_body(g, carry):
                for b in range(_B):
                    a = acc_v[b, pl.ds(g * 16, 16)]
                    acc_v[b, pl.ds(g * 16, 16)] = jnp.where(a > 0, one, -one)
                return carry

            lax.fori_loop(0, _BLK // 16, sgn_body, 0)

            pltpu.sync_copy(acc_v, out_hbm.at[:, pl.ds(c0, _BLK)])

        return carry

    lax.fori_loop(0, 3, blk_body, 0)


@jax.jit
def kernel(x, position_weight, level_weight):
    flat = x.reshape(_B, _N).T.reshape(-1)
    tail = _D - (_NBLK - 1) * _BLK
    p_tail = jnp.pad(position_weight[:, _D - tail:], ((0, 0), (0, _BLK - tail)))
    lw_tail = jnp.pad(level_weight[:, _D - tail:], ((0, 0), (0, _BLK - tail)))
    mesh = plsc.VectorSubcoreMesh(core_axis_name="c", subcore_axis_name="s")
    f = pl.kernel(
        _sc_body,
        out_type=jax.ShapeDtypeStruct((_B, _NBLK * _BLK), jnp.float32),
        mesh=mesh,
        scratch_types=[
            pltpu.VMEM((_B * _N,), jnp.float32),
            pltpu.VMEM((_B * _N,), jnp.int32),
            pltpu.VMEM((_L, _BLK), jnp.float32),
            pltpu.VMEM((_NC, _BLK), jnp.float32),
            pltpu.VMEM((_B, _BLK), jnp.float32),
        ],
    )
    out = f(flat, position_weight, level_weight, p_tail, lw_tail)
    return out[:, :_D]
